# baseline (speedup 1.0000x reference)
import jax
import jax.numpy as jnp
from jax.experimental import pallas as pl
from jax.experimental.pallas import tpu as pltpu
from jax.experimental.pallas import tpu_sc as plsc

_VOCAB = 100000
_EMBED = 64
_BATCH = 1024
_CTX = 20

_GW = 128
_VT = 2048
_NV = (_VOCAB + _VT - 1) // _VT


_NW = 32
_BPW = (_BATCH * _CTX) // _NW


def _sc_gather(table, flat_idx):
    n = _BATCH * _CTX
    mesh = plsc.VectorSubcoreMesh(core_axis_name="c", subcore_axis_name="s")

    @pl.kernel(out_type=jax.ShapeDtypeStruct((n, _EMBED), table.dtype),
               mesh=mesh,
               compiler_params=pltpu.CompilerParams(use_tc_tiling_on_sc=False),
               scratch_types=[
                   pltpu.VMEM((_BPW,), jnp.int32),
                   pltpu.VMEM((_BPW, _EMBED), jnp.float32),
                   pltpu.SemaphoreType.DMA,
               ])
    def gather_kernel(table_hbm, idx_hbm, out_hbm, idx_v, rows_v, sem):
        wid = jax.lax.axis_index("s") * 2 + jax.lax.axis_index("c")
        base = wid * _BPW
        pltpu.sync_copy(idx_hbm.at[pl.ds(base, _BPW)], idx_v)
        pltpu.async_copy(table_hbm.at[idx_v], rows_v, sem).wait()
        pltpu.sync_copy(rows_v, out_hbm.at[pl.ds(base, _BPW)])

    return gather_kernel(table, flat_idx)


def _project_body(emb_full_ref, w_ref, b_ref, out_ref, emb_scratch):
    j = pl.program_id(0)

    @pl.when(j == 0)
    def _():
        acc = emb_full_ref[pl.ds(0, _BATCH), :]
        for c in range(1, _CTX):
            acc = acc + emb_full_ref[pl.ds(c * _BATCH, _BATCH), :]
        emb_scratch[...] = acc * (1.0 / _CTX)

    out_ref[...] = jax.lax.dot_general(
        emb_scratch[...], w_ref[...],
        dimension_numbers=(((1,), (1,)), ((), ())),
        preferred_element_type=jnp.float32,
        precision=jax.lax.Precision.HIGHEST,
    ) + b_ref[...]


def _project(emb_full, W, b2):
    return pl.pallas_call(
        _project_body,
        grid=(_NV,),
        in_specs=[
            pl.BlockSpec((_BATCH * _CTX, _EMBED), lambda j: (0, 0)),
            pl.BlockSpec((_VT, _EMBED), lambda j: (j, 0)),
            pl.BlockSpec((1, _VT), lambda j: (0, j)),
        ],
        out_specs=pl.BlockSpec((_BATCH, _VT), lambda j: (0, j)),
        out_shape=jax.ShapeDtypeStruct((_BATCH, _VOCAB), jnp.float32),
        scratch_shapes=[pltpu.VMEM((_BATCH, _EMBED), jnp.float32)],
    )(emb_full, W, b2)


def kernel(inputs, table, W, b):
    flat_idx = inputs.T.reshape(_BATCH * _CTX).astype(jnp.int32)
    emb_full = _sc_gather(table, flat_idx)
    return _project(emb_full, W, b.reshape(1, _VOCAB))


# speedup vs baseline: 1.3015x; 1.3015x over previous
import jax
import jax.numpy as jnp
from jax.experimental import pallas as pl
from jax.experimental.pallas import tpu as pltpu
from jax.experimental.pallas import tpu_sc as plsc

_VOCAB = 100000
_EMBED = 64
_BATCH = 1024
_CTX = 20

_GW = 128
_VT = 2048
_NV = (_VOCAB + _VT - 1) // _VT


_NW = 32
_BPW = (_BATCH * _CTX) // _NW


def _sc_gather(table, flat_idx):
    n = _BATCH * _CTX
    mesh = plsc.VectorSubcoreMesh(core_axis_name="c", subcore_axis_name="s")

    @pl.kernel(out_type=jax.ShapeDtypeStruct((n, _EMBED), table.dtype),
               mesh=mesh,
               compiler_params=pltpu.CompilerParams(use_tc_tiling_on_sc=False),
               scratch_types=[
                   pltpu.VMEM((_BPW,), jnp.int32),
                   pltpu.VMEM((_BPW, _EMBED), jnp.float32),
                   pltpu.SemaphoreType.DMA,
               ])
    def gather_kernel(table_hbm, idx_hbm, out_hbm, idx_v, rows_v, sem):
        wid = jax.lax.axis_index("s") * 2 + jax.lax.axis_index("c")
        base = wid * _BPW
        pltpu.sync_copy(idx_hbm.at[pl.ds(base, _BPW)], idx_v)
        pltpu.async_copy(table_hbm.at[idx_v], rows_v, sem).wait()
        pltpu.sync_copy(rows_v, out_hbm.at[pl.ds(base, _BPW)])

    return gather_kernel(table, flat_idx)


def _project_body(emb_full_ref, w_ref, b_ref, out_ref, emb_scratch):
    j = pl.program_id(0)

    @pl.when(j == 0)
    def _():
        acc = emb_full_ref[pl.ds(0, _BATCH), :]
        for c in range(1, _CTX):
            acc = acc + emb_full_ref[pl.ds(c * _BATCH, _BATCH), :]
        emb_scratch[...] = (acc * (1.0 / _CTX)).astype(jnp.bfloat16)

    out_ref[...] = jax.lax.dot_general(
        emb_scratch[...], w_ref[...].astype(jnp.bfloat16),
        dimension_numbers=(((1,), (1,)), ((), ())),
        preferred_element_type=jnp.float32,
    ) + b_ref[...]


def _project(emb_full, W, b2):
    return pl.pallas_call(
        _project_body,
        grid=(_NV,),
        in_specs=[
            pl.BlockSpec((_BATCH * _CTX, _EMBED), lambda j: (0, 0)),
            pl.BlockSpec((_VT, _EMBED), lambda j: (j, 0)),
            pl.BlockSpec((1, _VT), lambda j: (0, j)),
        ],
        out_specs=pl.BlockSpec((_BATCH, _VT), lambda j: (0, j)),
        out_shape=jax.ShapeDtypeStruct((_BATCH, _VOCAB), jnp.float32),
        scratch_shapes=[pltpu.VMEM((_BATCH, _EMBED), jnp.bfloat16)],
    )(emb_full, W, b2)


def kernel(inputs, table, W, b):
    flat_idx = inputs.T.reshape(_BATCH * _CTX).astype(jnp.int32)
    emb_full = _sc_gather(table, flat_idx)
    return _project(emb_full, W, b.reshape(1, _VOCAB))


# speedup vs baseline: 1.3143x; 1.0099x over previous
import jax
import jax.numpy as jnp
from jax.experimental import pallas as pl
from jax.experimental.pallas import tpu as pltpu
from jax.experimental.pallas import tpu_sc as plsc

_VOCAB = 100000
_EMBED = 64
_BATCH = 1024
_CTX = 20

_GW = 128
_VT = 2048
_NV = (_VOCAB + _VT - 1) // _VT


_NW = 32
_BPW = (_BATCH * _CTX) // _NW


def _sc_gather(table, flat_idx):
    n = _BATCH * _CTX
    mesh = plsc.VectorSubcoreMesh(core_axis_name="c", subcore_axis_name="s")

    @pl.kernel(out_type=jax.ShapeDtypeStruct((n, _EMBED), table.dtype),
               mesh=mesh,
               compiler_params=pltpu.CompilerParams(use_tc_tiling_on_sc=False),
               scratch_types=[
                   pltpu.VMEM((_BPW,), jnp.int32),
                   pltpu.VMEM((_BPW, _EMBED), jnp.float32),
                   pltpu.SemaphoreType.DMA,
               ])
    def gather_kernel(table_hbm, idx_hbm, out_hbm, idx_v, rows_v, sem):
        wid = jax.lax.axis_index("s") * 2 + jax.lax.axis_index("c")
        base = wid * _BPW
        pltpu.sync_copy(idx_hbm.at[pl.ds(base, _BPW)], idx_v)
        pltpu.async_copy(table_hbm.at[idx_v], rows_v, sem).wait()
        pltpu.sync_copy(rows_v, out_hbm.at[pl.ds(base, _BPW)])

    return gather_kernel(table, flat_idx)


def _mean_body(emb_full_ref, emb_ref):
    acc = emb_full_ref[pl.ds(0, _BATCH), :]
    for c in range(1, _CTX):
        acc = acc + emb_full_ref[pl.ds(c * _BATCH, _BATCH), :]
    emb_ref[...] = (acc * (1.0 / _CTX)).astype(jnp.bfloat16)


def _mean(emb_full):
    return pl.pallas_call(
        _mean_body,
        out_shape=jax.ShapeDtypeStruct((_BATCH, _EMBED), jnp.bfloat16),
    )(emb_full)


def _project_body(emb_ref, w_ref, b_ref, out_ref):
    out_ref[...] = jax.lax.dot_general(
        emb_ref[...], w_ref[...].astype(jnp.bfloat16),
        dimension_numbers=(((1,), (1,)), ((), ())),
        preferred_element_type=jnp.float32,
    ) + b_ref[...]


def _project(emb, W, b2):
    return pl.pallas_call(
        _project_body,
        grid=(_NV,),
        in_specs=[
            pl.BlockSpec((_BATCH, _EMBED), lambda j: (0, 0)),
            pl.BlockSpec((_VT, _EMBED), lambda j: (j, 0)),
            pl.BlockSpec((1, _VT), lambda j: (0, j)),
        ],
        out_specs=pl.BlockSpec((_BATCH, _VT), lambda j: (0, j)),
        out_shape=jax.ShapeDtypeStruct((_BATCH, _VOCAB), jnp.float32),
        compiler_params=pltpu.CompilerParams(
            dimension_semantics=("arbitrary",)),
    )(emb, W, b2)


def kernel(inputs, table, W, b):
    flat_idx = inputs.T.reshape(_BATCH * _CTX).astype(jnp.int32)
    emb_full = _sc_gather(table, flat_idx)
    emb = _mean(emb_full)
    return _project(emb, W, b.reshape(1, _VOCAB))
